# Initial kernel scaffold; baseline (speedup 1.0000x reference)
import jax
import jax.numpy as jnp
from jax.experimental import pallas as pl


def kernel(x, edge_index, edge_weight, W1, b1, Wc, bc, W2, b2):
    raise NotImplementedError("write your pallas kernel here")


# speedup vs baseline: 1.3619x; 1.3619x over previous
import jax
import jax.numpy as jnp
from jax.experimental import pallas as pl

N = 10000
E = 320000
ALPHA = 0.1
K = 10


def _mlp_kernel(x_ref, w1_ref, b1_ref, wc_ref, o_ref):
    h = jnp.dot(x_ref[...], w1_ref[...], preferred_element_type=jnp.float32)
    h = jax.nn.relu(h + b1_ref[...])
    o_ref[...] = jnp.dot(h, wc_ref[...], preferred_element_type=jnp.float32)


def _fused_mlp(x, W1, b1, Wc):
    F_IN = x.shape[1]
    HID = W1.shape[1]
    BR = 1000
    return pl.pallas_call(
        _mlp_kernel,
        grid=(N // BR,),
        in_specs=[
            pl.BlockSpec((BR, F_IN), lambda i: (i, 0)),
            pl.BlockSpec((F_IN, HID), lambda i: (0, 0)),
            pl.BlockSpec((1, HID), lambda i: (0, 0)),
            pl.BlockSpec((HID, HID), lambda i: (0, 0)),
        ],
        out_specs=pl.BlockSpec((BR, HID), lambda i: (i, 0)),
        out_shape=jax.ShapeDtypeStruct((N, HID), jnp.float32),
    )(x, W1, b1.reshape(1, HID), Wc)


def kernel(x, edge_index, edge_weight, W1, b1, Wc, bc, W2, b2):
    row = edge_index[0]
    col = edge_index[1]
    deg = jnp.ones((N,), jnp.float32).at[col].add(edge_weight)
    dinv = jax.lax.rsqrt(deg)
    norm = dinv[row] * edge_weight * dinv[col]
    dself = dinv * dinv

    m = _fused_mlp(x, W1, b1, Wc)

    def prop(h):
        out = h * dself[:, None]
        msg = norm[:, None] * h[row]
        return out.at[col].add(msg)

    h = jax.nn.relu(prop(m) + bc)
    h = h @ W2 + b2
    x0 = h
    for _ in range(K):
        h = prop(h) * (1.0 - ALPHA) + ALPHA * x0
    return jax.nn.log_softmax(h, axis=-1)
